# baseline (speedup 1.0000x reference)
import functools

import jax
import jax.numpy as jnp
from jax import lax
from jax.experimental import pallas as pl
from jax.experimental.pallas import tpu as pltpu
from jax.experimental.pallas import tpu_sc as plsc

BATCH = 16384
N_FIELDS = 26
VOCAB = 100000
EMBED = 16
N_CONTI = 74
OUT_W = N_FIELDS * EMBED + N_CONTI
OUT_WP = 496
N_CTILE = OUT_WP // 8
N_BTILE = BATCH // 128

VOCAB_PAD = 100352

NC, NS, L = 2, 16, 16
NW = NC * NS
ROWS_PER_W = BATCH // NW
C = 128
N_CHUNKS = ROWS_PER_W // C
G = C // L
N_BLOCKS = N_CTILE // 2
DEPTH = 4


def _body(x_hbm, w_hbm, out_hbm, x_buf, idx_buf, emb_buf, stage,
          xsem, gsem, ssem):
    wid = lax.axis_index("s") * NC + lax.axis_index("c")
    iota = lax.iota(jnp.int32, L)
    zeros = jnp.zeros((L,), jnp.float32)
    e_consts = [jnp.full((L,), e, jnp.int32) for e in range(EMBED)]

    def x_copy(t, slot):
        cb = wid * ROWS_PER_W + t * C
        return pltpu.make_async_copy(
            x_hbm.at[:, pl.ds(cb, C)], x_buf.at[slot], xsem)

    def gather(f, slot):
        return pltpu.make_async_copy(
            w_hbm.at[idx_buf.at[f]], emb_buf.at[slot], gsem)

    def stage_dma(u, slot, bt):
        return pltpu.make_async_copy(
            stage.at[slot], out_hbm.at[pl.ds(2 * u, 2), bt], ssem)

    x_copy(0, 0).start()

    def chunk(t, _):
        xslot = lax.rem(t, 2)
        bt = wid * N_CHUNKS + t

        x_copy(t, xslot).wait()

        @pl.when(t + 1 < N_CHUNKS)
        def _():
            x_copy(t + 1, 1 - xslot).start()

        def field_idx(f, _):
            base = lax.div(f, 8) * (782 * 1024) + lax.rem(f, 8)
            for g in range(G):
                c = x_buf[xslot, f, pl.ds(g * L, L)]
                r = ((c >> 7) << 10) + ((c & 127) << 3)
                idx_buf[f, pl.ds(g * L, L)] = r + base
            return 0

        lax.fori_loop(0, N_FIELDS, field_idx, 0)

        for f in range(N_FIELDS):
            gather(f, f).start()

        def block(u, _):
            s = lax.rem(u, DEPTH)

            @pl.when(u >= DEPTH)
            def _():
                stage_dma(u - DEPTH, s, bt).wait()

            @pl.when(u < N_FIELDS)
            def _():
                gather(u, u).wait()

                u_splat = jnp.full((L,), u, jnp.int32)
                for g in range(G):
                    rows = g * L + iota
                    for e in range(EMBED):
                        vals = plsc.load_gather(
                            emb_buf, [u_splat, rows, e_consts[e]])
                        stage[s, e // 8, e % 8, pl.ds(g * L, L)] = vals

            @pl.when(u >= N_FIELDS)
            def _():
                q = u - N_FIELDS
                for h in range(2):
                    for r in range(8):
                        j = q * 16 + h * 8 + r
                        for g in range(G):
                            @pl.when(j < N_CONTI)
                            def _():
                                vals = x_buf[xslot, N_FIELDS + j,
                                             pl.ds(g * L, L)]
                                stage[s, h, r, pl.ds(g * L, L)] = (
                                    vals.astype(jnp.float32))

                            @pl.when(j >= N_CONTI)
                            def _():
                                stage[s, h, r, pl.ds(g * L, L)] = zeros

            stage_dma(u, s, bt).start()
            return 0

        lax.fori_loop(0, N_BLOCKS, block, 0)

        def drain(u, _):
            stage_dma(u, lax.rem(u, DEPTH), bt).wait()
            return 0

        lax.fori_loop(N_BLOCKS - DEPTH, N_BLOCKS, drain, 0)
        return 0

    lax.fori_loop(0, N_CHUNKS, chunk, 0)


def _transpose_body(wt_ref, out_ref):
    blk = wt_ref[...].reshape(128, 4096)
    for k in range(32):
        out_ref[0, k] = blk[:, 128 * k:128 * (k + 1)].T


@jax.jit
def kernel(x, W):
    wt = W.transpose(0, 2, 1)
    w4 = pl.pallas_call(
        _transpose_body,
        grid=(4, 25),
        in_specs=[pl.BlockSpec((8, 16, 4096), lambda fg, cb: (fg, 0, cb))],
        out_specs=pl.BlockSpec((1, 32, 128, 128),
                               lambda fg, cb: (fg, cb, 0, 0)),
        out_shape=jax.ShapeDtypeStruct((4, 782, 128, 128), jnp.float32),
    )(wt)
    w_flat = w4.reshape(4 * 782 * 128 * 8, EMBED)
    xt = x.T

    run = functools.partial(
        pl.kernel,
        out_type=jax.ShapeDtypeStruct((N_CTILE, N_BTILE, 8, 128),
                                      jnp.float32),
        mesh=plsc.VectorSubcoreMesh(core_axis_name="c", subcore_axis_name="s"),
        compiler_params=pltpu.CompilerParams(
            use_tc_tiling_on_sc=False, needs_layout_passes=False),
        scratch_types=[
            pltpu.VMEM((2, 100, C), jnp.int32),
            pltpu.VMEM((N_FIELDS, C), jnp.int32),
            pltpu.VMEM((N_FIELDS, C, EMBED), jnp.float32),
            pltpu.VMEM((DEPTH, 2, 8, C), jnp.float32),
            pltpu.SemaphoreType.DMA,
            pltpu.SemaphoreType.DMA,
            pltpu.SemaphoreType.DMA,
        ],
    )(_body)
    out3 = run(xt, w_flat)
    return out3.transpose(1, 3, 0, 2).reshape(BATCH, OUT_WP)[:, :OUT_W]


# speedup vs baseline: 1.3426x; 1.3426x over previous
import functools

import jax
import jax.numpy as jnp
from jax import lax
from jax.experimental import pallas as pl
from jax.experimental.pallas import tpu as pltpu
from jax.experimental.pallas import tpu_sc as plsc

BATCH = 16384
N_FIELDS = 26
VOCAB = 100000
EMBED = 16
N_CONTI = 74
OUT_W = N_FIELDS * EMBED + N_CONTI
OUT_WP = 496
N_CTILE = OUT_WP // 8
N_BTILE = BATCH // 128

VOCAB_PAD = 100352

NC, NS, L = 2, 16, 16
NW = NC * NS
ROWS_PER_W = BATCH // NW
C = 128
N_CHUNKS = ROWS_PER_W // C
G = C // L
N_BLOCKS = N_CTILE // 2
DEPTH = 4


def _body(x_hbm, w_hbm, out_hbm, x_buf, idx_buf, emb_buf, stage,
          xsem, gsem, ssem):
    wid = lax.axis_index("s") * NC + lax.axis_index("c")
    iota = lax.iota(jnp.int32, L)
    zeros = jnp.zeros((L,), jnp.float32)
    e_consts = [jnp.full((L,), e, jnp.int32) for e in range(EMBED)]

    def x_copy(t, slot):
        cb = wid * ROWS_PER_W + t * C
        return pltpu.make_async_copy(
            x_hbm.at[:, pl.ds(cb, C)], x_buf.at[slot], xsem)

    def gather(f, slot):
        return pltpu.make_async_copy(
            w_hbm.at[idx_buf.at[f]], emb_buf.at[slot], gsem)

    def stage_dma(u, slot, bt):
        return pltpu.make_async_copy(
            stage.at[slot], out_hbm.at[pl.ds(2 * u, 2), bt], ssem)

    x_copy(0, 0).start()

    def chunk(t, _):
        xslot = lax.rem(t, 2)
        bt = wid * N_CHUNKS + t

        x_copy(t, xslot).wait()

        @pl.when(t + 1 < N_CHUNKS)
        def _():
            x_copy(t + 1, 1 - xslot).start()

        def field_idx(f, _):
            base = lax.div(f, 8) * (782 * 1024) + lax.rem(f, 8)
            for g in range(G):
                c = x_buf[xslot, f, pl.ds(g * L, L)]
                r = ((c >> 7) << 10) + ((c & 127) << 3)
                idx_buf[f, pl.ds(g * L, L)] = r + base
            return 0

        lax.fori_loop(0, N_FIELDS, field_idx, 0)

        for f in range(N_FIELDS):
            gather(f, f).start()

        def block(u, _):
            s = lax.rem(u, DEPTH)

            @pl.when(u >= DEPTH)
            def _():
                stage_dma(u - DEPTH, s, bt).wait()

            @pl.when(u < N_FIELDS)
            def _():
                gather(u, u).wait()

                u_splat = jnp.full((L,), u, jnp.int32)
                for g in range(G):
                    rows = g * L + iota
                    vals = [plsc.load_gather(
                        emb_buf, [u_splat, rows, e_consts[e]])
                        for e in range(EMBED)]
                    for e in range(EMBED):
                        stage[s, e // 8, e % 8, pl.ds(g * L, L)] = vals[e]

            @pl.when(u >= N_FIELDS)
            def _():
                q = u - N_FIELDS
                for h in range(2):
                    for r in range(8):
                        j = q * 16 + h * 8 + r

                        @pl.when(j < N_CONTI)
                        def _():
                            vals = [x_buf[xslot, N_FIELDS + j,
                                          pl.ds(g * L, L)].astype(
                                              jnp.float32)
                                    for g in range(G)]
                            for g in range(G):
                                stage[s, h, r, pl.ds(g * L, L)] = vals[g]

                        @pl.when(j >= N_CONTI)
                        def _():
                            for g in range(G):
                                stage[s, h, r, pl.ds(g * L, L)] = zeros

            stage_dma(u, s, bt).start()
            return 0

        lax.fori_loop(0, N_BLOCKS, block, 0)

        def drain(u, _):
            stage_dma(u, lax.rem(u, DEPTH), bt).wait()
            return 0

        lax.fori_loop(N_BLOCKS - DEPTH, N_BLOCKS, drain, 0)
        return 0

    lax.fori_loop(0, N_CHUNKS, chunk, 0)


def _transpose_body(wt_ref, out_ref):
    blk = wt_ref[...].reshape(128, 4096)
    for k in range(32):
        out_ref[0, k] = blk[:, 128 * k:128 * (k + 1)].T


@jax.jit
def kernel(x, W):
    wt = W.transpose(0, 2, 1)
    w4 = pl.pallas_call(
        _transpose_body,
        grid=(4, 25),
        in_specs=[pl.BlockSpec((8, 16, 4096), lambda fg, cb: (fg, 0, cb))],
        out_specs=pl.BlockSpec((1, 32, 128, 128),
                               lambda fg, cb: (fg, cb, 0, 0)),
        out_shape=jax.ShapeDtypeStruct((4, 782, 128, 128), jnp.float32),
    )(wt)
    w_flat = w4.reshape(4 * 782 * 128 * 8, EMBED)
    xt = x.T

    run = functools.partial(
        pl.kernel,
        out_type=jax.ShapeDtypeStruct((N_CTILE, N_BTILE, 8, 128),
                                      jnp.float32),
        mesh=plsc.VectorSubcoreMesh(core_axis_name="c", subcore_axis_name="s"),
        compiler_params=pltpu.CompilerParams(
            use_tc_tiling_on_sc=False, needs_layout_passes=False),
        scratch_types=[
            pltpu.VMEM((2, 100, C), jnp.int32),
            pltpu.VMEM((N_FIELDS, C), jnp.int32),
            pltpu.VMEM((N_FIELDS, C, EMBED), jnp.float32),
            pltpu.VMEM((DEPTH, 2, 8, C), jnp.float32),
            pltpu.SemaphoreType.DMA,
            pltpu.SemaphoreType.DMA,
            pltpu.SemaphoreType.DMA,
        ],
    )(_body)
    out3 = run(xt, w_flat)
    return out3.transpose(1, 3, 0, 2).reshape(BATCH, OUT_WP)[:, :OUT_W]
